# baseline (speedup 1.0000x reference)
import functools

import jax
import jax.numpy as jnp
from jax import lax
from jax.experimental import pallas as pl
from jax.experimental.pallas import tpu as pltpu
from jax.experimental.pallas import tpu_sc as plsc

_NB = 256


def _argmin_kernel(pred_ref, realt_ref, out_ref):
    m = realt_ref.shape[2]
    p = pred_ref[0]
    rt = realt_ref[0]
    q2 = jnp.sum(p * p, axis=1, keepdims=True)
    k2 = jnp.sum(rt * rt, axis=0, keepdims=True)
    nqk2 = jax.lax.dot_general(
        -2.0 * p, rt, (((1,), (0,)), ((), ())),
        preferred_element_type=jnp.float32,
    )
    val = jnp.maximum(q2 + nqk2 + k2, 0.0)
    minv = jnp.min(val, axis=1, keepdims=True)
    iota = lax.broadcasted_iota(jnp.int32, val.shape, 1)
    idx = jnp.min(jnp.where(val == minv, iota, m), axis=1)
    out_ref[0, 0, :] = idx + pl.program_id(0) * m


def _nearest_indices(predicted_positions, real_positions):
    B, N, d = predicted_positions.shape
    M = real_positions.shape[1]
    realt = jnp.transpose(real_positions, (0, 2, 1))
    idx3 = pl.pallas_call(
        _argmin_kernel,
        grid=(B, N // _NB),
        in_specs=[
            pl.BlockSpec((1, _NB, d), lambda b, n: (b, n, 0)),
            pl.BlockSpec((1, d, M), lambda b, n: (b, 0, 0)),
        ],
        out_specs=pl.BlockSpec((1, 1, _NB), lambda b, n: (b, 0, n)),
        out_shape=jax.ShapeDtypeStruct((B, 1, N), jnp.int32),
        compiler_params=pltpu.CompilerParams(
            dimension_semantics=("arbitrary", "arbitrary"),
        ),
    )(predicted_positions, realt)
    return idx3.reshape(B * N)


def _make_sc_gather(R, V, G, n_chunks, chunk):
    mesh = plsc.VectorSubcoreMesh(core_axis_name="c", subcore_axis_name="s")
    info = plsc.get_sparse_core_info()
    nc = info.num_cores
    rows_per_w = R // (nc * info.num_subcores)

    @functools.partial(
        pl.kernel,
        mesh=mesh,
        out_type=jax.ShapeDtypeStruct((R, G), jnp.float32),
        scratch_types=[
            pltpu.VMEM((rows_per_w,), jnp.int32),
            pltpu.VMEM((chunk, G), jnp.float32),
            pltpu.VMEM((chunk, G), jnp.float32),
            pltpu.SemaphoreType.DMA,
            pltpu.SemaphoreType.DMA,
            pltpu.SemaphoreType.DMA,
            pltpu.SemaphoreType.DMA,
        ],
    )
    def gather(table_hbm, idx_hbm, out_hbm, idx_v, buf0, buf1, g0, g1, s0, s1):
        wid = lax.axis_index("s") * nc + lax.axis_index("c")
        base = wid * rows_per_w
        pltpu.sync_copy(idx_hbm.at[pl.ds(base, rows_per_w)], idx_v)
        bufs = (buf0, buf1)
        gsems = (g0, g1)
        osems = (s0, s1)
        copies = [None] * n_chunks
        outs = [None] * n_chunks

        def start(c):
            copies[c] = pltpu.make_async_copy(
                table_hbm.at[idx_v.at[pl.ds(c * chunk, chunk)]],
                bufs[c % 2],
                gsems[c % 2],
            )
            copies[c].start()

        start(0)
        for c in range(n_chunks):
            copies[c].wait()
            if c + 1 < n_chunks:
                if c >= 1:
                    outs[c - 1].wait()
                start(c + 1)
            outs[c] = pltpu.make_async_copy(
                bufs[c % 2],
                out_hbm.at[pl.ds(base + c * chunk, chunk)],
                osems[c % 2],
            )
            outs[c].start()
        outs[n_chunks - 2].wait()
        outs[n_chunks - 1].wait()

    return gather


def kernel(predicted_positions, real_positions, real_expressions):
    B, N, d = predicted_positions.shape
    M = real_positions.shape[1]
    G = real_expressions.shape[2]
    flat_idx = _nearest_indices(predicted_positions, real_positions)
    table = real_expressions.reshape(B * M, G)
    gather = _make_sc_gather(B * N, B * M, G, n_chunks=8, chunk=64)
    out = gather(table, flat_idx)
    return out.reshape(B, N, G)


# speedup vs baseline: 1.0989x; 1.0989x over previous
import functools

import jax
import jax.numpy as jnp
from jax import lax
from jax.experimental import pallas as pl
from jax.experimental.pallas import tpu as pltpu
from jax.experimental.pallas import tpu_sc as plsc

_NB = 256


def _argmin_kernel(pred_ref, realt_ref, out_ref):
    m = realt_ref.shape[2]
    p = pred_ref[0]
    rt = realt_ref[0]
    q2 = jnp.sum(p * p, axis=1, keepdims=True)
    k2 = jnp.sum(rt * rt, axis=0, keepdims=True)
    nqk2 = jax.lax.dot_general(
        -2.0 * p, rt, (((1,), (0,)), ((), ())),
        preferred_element_type=jnp.float32,
    )
    sq = (q2 + nqk2) + k2
    minv = jnp.min(sq, axis=1, keepdims=True)
    t = jnp.maximum(minv, 0.0)
    fiota = lax.broadcasted_iota(jnp.int32, (1, m), 1).astype(jnp.float32)
    fidx = jnp.min(jnp.where(sq <= t, fiota, float(m)), axis=1)
    out_ref[0, 0, :] = fidx.astype(jnp.int32) + pl.program_id(0) * m


def _nearest_indices(predicted_positions, real_positions):
    B, N, d = predicted_positions.shape
    M = real_positions.shape[1]
    realt = jnp.transpose(real_positions, (0, 2, 1))
    idx3 = pl.pallas_call(
        _argmin_kernel,
        grid=(B, N // _NB),
        in_specs=[
            pl.BlockSpec((1, _NB, d), lambda b, n: (b, n, 0)),
            pl.BlockSpec((1, d, M), lambda b, n: (b, 0, 0)),
        ],
        out_specs=pl.BlockSpec((1, 1, _NB), lambda b, n: (b, 0, n)),
        out_shape=jax.ShapeDtypeStruct((B, 1, N), jnp.int32),
        compiler_params=pltpu.CompilerParams(
            dimension_semantics=("arbitrary", "arbitrary"),
        ),
    )(predicted_positions, realt)
    return idx3.reshape(B * N)


def _make_sc_gather(R, V, G, n_chunks, chunk):
    mesh = plsc.VectorSubcoreMesh(core_axis_name="c", subcore_axis_name="s")
    info = plsc.get_sparse_core_info()
    nc = info.num_cores
    rows_per_w = R // (nc * info.num_subcores)

    @functools.partial(
        pl.kernel,
        mesh=mesh,
        out_type=jax.ShapeDtypeStruct((R, G), jnp.float32),
        scratch_types=[
            pltpu.VMEM((rows_per_w,), jnp.int32),
            pltpu.VMEM((chunk, G), jnp.float32),
            pltpu.VMEM((chunk, G), jnp.float32),
            pltpu.SemaphoreType.DMA,
            pltpu.SemaphoreType.DMA,
            pltpu.SemaphoreType.DMA,
            pltpu.SemaphoreType.DMA,
        ],
    )
    def gather(table_hbm, idx_hbm, out_hbm, idx_v, buf0, buf1, g0, g1, s0, s1):
        wid = lax.axis_index("s") * nc + lax.axis_index("c")
        base = wid * rows_per_w
        pltpu.sync_copy(idx_hbm.at[pl.ds(base, rows_per_w)], idx_v)
        bufs = (buf0, buf1)
        gsems = (g0, g1)
        osems = (s0, s1)
        copies = [None] * n_chunks
        outs = [None] * n_chunks

        def start(c):
            copies[c] = pltpu.make_async_copy(
                table_hbm.at[idx_v.at[pl.ds(c * chunk, chunk)]],
                bufs[c % 2],
                gsems[c % 2],
            )
            copies[c].start()

        start(0)
        for c in range(n_chunks):
            copies[c].wait()
            if c + 1 < n_chunks:
                if c >= 1:
                    outs[c - 1].wait()
                start(c + 1)
            outs[c] = pltpu.make_async_copy(
                bufs[c % 2],
                out_hbm.at[pl.ds(base + c * chunk, chunk)],
                osems[c % 2],
            )
            outs[c].start()
        outs[n_chunks - 2].wait()
        outs[n_chunks - 1].wait()

    return gather


def kernel(predicted_positions, real_positions, real_expressions):
    B, N, d = predicted_positions.shape
    M = real_positions.shape[1]
    G = real_expressions.shape[2]
    flat_idx = _nearest_indices(predicted_positions, real_positions)
    table = real_expressions.reshape(B * M, G)
    gather = _make_sc_gather(B * N, B * M, G, n_chunks=8, chunk=64)
    out = gather(table, flat_idx)
    return out.reshape(B, N, G)
